# baseline (speedup 1.0000x reference)
import jax
import jax.numpy as jnp
from jax import lax
from jax.experimental import pallas as pl
from jax.experimental.pallas import tpu as pltpu


def _dist_body(x_ref, w_ref, dist_ref, t_ref):
    x = x_ref[...]
    w = w_ref[...]
    x2 = jnp.sum(x * x, axis=1, keepdims=True)
    w2 = jnp.sum(w * w, axis=1)
    xw = lax.dot_general(x, w, (((1,), (1,)), ((), ())),
                         preferred_element_type=jnp.float32)
    dist = x2 + w2[None, :] - 2.0 * xw
    dist_ref[...] = dist
    neg = -dist
    m = jnp.max(neg, axis=1, keepdims=True)
    s = jnp.sum(jnp.exp(neg - m), axis=1, keepdims=True)
    t_ref[...] = m + jnp.log(s)


def _assign_body(dist_ref, t_ref, w_ref, assign_ref, qT_ref):
    d = dist_ref[...]
    t = t_ref[...]
    p = jnp.exp(-(d + t))
    pT = jnp.transpose(p)
    assign_ref[...] = pT[None]
    qT_part = lax.dot_general(w_ref[...], pT, (((0,), (0,)), ((), ())),
                              preferred_element_type=jnp.float32)

    @pl.when(pl.program_id(1) == 0)
    def _init():
        qT_ref[...] = qT_part[None]

    @pl.when(pl.program_id(1) > 0)
    def _acc():
        qT_ref[...] += qT_part[None]


def kernel(feat, vq_weight):
    b, c, h, w = feat.shape
    k = vq_weight.shape[0]
    n_per_b = h * w
    n = b * n_per_b
    r_tile = 192
    nr = n // r_tile
    k_tile = 2048
    nk = k // k_tile
    flat = jnp.transpose(feat, (0, 2, 3, 1)).reshape(n, c)

    dist, t = pl.pallas_call(
        _dist_body,
        grid=(nr,),
        in_specs=[
            pl.BlockSpec((r_tile, c), lambda i: (i, 0)),
            pl.BlockSpec((k, c), lambda i: (0, 0)),
        ],
        out_specs=[
            pl.BlockSpec((r_tile, k), lambda i: (i, 0)),
            pl.BlockSpec((r_tile, 1), lambda i: (i, 0)),
        ],
        out_shape=[
            jax.ShapeDtypeStruct((n, k), jnp.float32),
            jax.ShapeDtypeStruct((n, 1), jnp.float32),
        ],
        compiler_params=pltpu.CompilerParams(
            dimension_semantics=("parallel",),
        ),
    )(flat, vq_weight)

    assign_f, qT_f = pl.pallas_call(
        _assign_body,
        grid=(b, nk),
        in_specs=[
            pl.BlockSpec((n_per_b, k_tile), lambda i, j: (i, j)),
            pl.BlockSpec((n_per_b, 1), lambda i, j: (i, 0)),
            pl.BlockSpec((k_tile, c), lambda i, j: (j, 0)),
        ],
        out_specs=[
            pl.BlockSpec((1, k_tile, n_per_b), lambda i, j: (i, j, 0)),
            pl.BlockSpec((1, c, n_per_b), lambda i, j: (i, 0, 0)),
        ],
        out_shape=[
            jax.ShapeDtypeStruct((b, k, n_per_b), jnp.float32),
            jax.ShapeDtypeStruct((b, c, n_per_b), jnp.float32),
        ],
        compiler_params=pltpu.CompilerParams(
            dimension_semantics=("parallel", "arbitrary"),
        ),
    )(dist, t, vq_weight)

    featp = flat.reshape(b, h, w, c)
    q_feat = qT_f.reshape(b, c, h, w)
    assignment = assign_f.reshape(b, k, h, w)
    return (featp, q_feat, assignment, dist)


# speedup vs baseline: 2.5947x; 2.5947x over previous
import jax
import jax.numpy as jnp
from jax import lax
from jax.experimental import pallas as pl
from jax.experimental.pallas import tpu as pltpu


def _vq_body(x_ref, w_ref, dist_ref, p_ref, q_ref, xout_ref):
    x = x_ref[...]
    w = w_ref[...]
    x2 = jnp.sum(x * x, axis=1, keepdims=True)
    w2 = jnp.sum(w * w, axis=1)
    xw = lax.dot_general(x, w, (((1,), (1,)), ((), ())),
                         preferred_element_type=jnp.float32)
    dist = x2 + w2[None, :] - 2.0 * xw
    dist_ref[...] = dist
    neg = -dist
    m = jnp.max(neg, axis=1, keepdims=True)
    e = jnp.exp(neg - m)
    s = jnp.sum(e, axis=1, keepdims=True)
    p = e / s
    p_ref[...] = p
    q_ref[...] = lax.dot_general(p, w, (((1,), (0,)), ((), ())),
                                 preferred_element_type=jnp.float32)
    xout_ref[...] = x


def kernel(feat, vq_weight):
    b, c, h, w = feat.shape
    k = vq_weight.shape[0]
    n = b * h * w
    r_tile = 192
    nr = n // r_tile
    flat = jnp.transpose(feat, (0, 2, 3, 1)).reshape(n, c)

    dist, p_flat, q, x_out = pl.pallas_call(
        _vq_body,
        grid=(nr,),
        in_specs=[
            pl.BlockSpec((r_tile, c), lambda i: (i, 0)),
            pl.BlockSpec((k, c), lambda i: (0, 0)),
        ],
        out_specs=[
            pl.BlockSpec((r_tile, k), lambda i: (i, 0)),
            pl.BlockSpec((r_tile, k), lambda i: (i, 0)),
            pl.BlockSpec((r_tile, c), lambda i: (i, 0)),
            pl.BlockSpec((r_tile, c), lambda i: (i, 0)),
        ],
        out_shape=[
            jax.ShapeDtypeStruct((n, k), jnp.float32),
            jax.ShapeDtypeStruct((n, k), jnp.float32),
            jax.ShapeDtypeStruct((n, c), jnp.float32),
            jax.ShapeDtypeStruct((n, c), jnp.float32),
        ],
        compiler_params=pltpu.CompilerParams(
            dimension_semantics=("parallel",),
        ),
    )(flat, vq_weight)

    featp = x_out.reshape(b, h, w, c)
    q_feat = jnp.transpose(q.reshape(b, h, w, c), (0, 3, 1, 2))
    assignment = jnp.transpose(p_flat.reshape(b, h, w, k), (0, 3, 1, 2))
    return (featp, q_feat, assignment, dist)
